# Initial kernel scaffold; baseline (speedup 1.0000x reference)
import jax
import jax.numpy as jnp
from jax.experimental import pallas as pl


def kernel(inputs, embedding_weight):
    raise NotImplementedError("write your pallas kernel here")


# speedup vs baseline: 1.3609x; 1.3609x over previous
import functools

import jax
import jax.numpy as jnp
from jax import lax
from jax.experimental import pallas as pl
from jax.experimental.pallas import tpu as pltpu
from jax.experimental.pallas import tpu_sc as plsc

_K = 8192
_D = 256
_N = 16384
_BN = 256
_STEPS = _N // _BN

_NC = 2
_NS = 16
_NW = _NC * _NS
_BPW = _N // _NW
_CH = 256


def _dist_kernel(x_ref, et_ref, xsq_ref, esq_ref, idx_ref, msum_ref):
    i = pl.program_id(0)
    xb = x_ref[...].astype(jnp.bfloat16)
    eb = et_ref[...].astype(jnp.bfloat16)
    dot = lax.dot_general(
        xb, eb, (((1,), (1,)), ((), ())),
        preferred_element_type=jnp.float32)
    d = (xsq_ref[...] - 2.0 * dot) + esq_ref[...]
    m = jnp.min(d, axis=1, keepdims=True)
    kio = lax.broadcasted_iota(jnp.int32, (_BN, _K), 1)
    cand = jnp.where(d == m, kio, _K)
    idx_ref[...] = jnp.min(cand, axis=1, keepdims=True)

    @pl.when(i == 0)
    def _():
        msum_ref[...] = jnp.zeros_like(msum_ref)

    msum_ref[...] += jnp.sum(m)


def _argmin_dist(flat, e_t, xsq, esq):
    return pl.pallas_call(
        _dist_kernel,
        grid=(_STEPS,),
        in_specs=[
            pl.BlockSpec((_BN, _D), lambda i: (i, 0)),
            pl.BlockSpec((_K, _D), lambda i: (0, 0)),
            pl.BlockSpec((_BN, 1), lambda i: (i, 0)),
            pl.BlockSpec((1, _K), lambda i: (0, 0)),
        ],
        out_specs=[
            pl.BlockSpec((_BN, 1), lambda i: (i, 0)),
            pl.BlockSpec((1, 1), lambda i: (0, 0)),
        ],
        out_shape=[
            jax.ShapeDtypeStruct((_N, 1), jnp.int32),
            jax.ShapeDtypeStruct((1, 1), jnp.float32),
        ],
    )(flat, e_t, xsq, esq)


def _sc_gather(table, idx):
    mesh = plsc.VectorSubcoreMesh(core_axis_name="c", subcore_axis_name="s")

    @functools.partial(
        pl.kernel,
        out_type=jax.ShapeDtypeStruct((_N, _D), jnp.float32),
        mesh=mesh,
        scratch_types=[
            pltpu.VMEM((_BPW,), jnp.int32),
            pltpu.VMEM((_CH, _D), jnp.float32),
            pltpu.SemaphoreType.DMA,
        ],
    )
    def gather_k(table_hbm, idx_hbm, out_hbm, idx_v, rows_v, sem):
        wid = lax.axis_index("s") * _NC + lax.axis_index("c")
        base = wid * _BPW
        pltpu.sync_copy(idx_hbm.at[pl.ds(base, _BPW)], idx_v)
        for c in range(_BPW // _CH):
            pltpu.async_copy(
                table_hbm.at[idx_v.at[pl.ds(c * _CH, _CH)]], rows_v, sem
            ).wait()
            pltpu.sync_copy(rows_v, out_hbm.at[pl.ds(base + c * _CH, _CH)])

    return gather_k(table, idx)


def kernel(inputs, embedding_weight):
    B, C, H, W = inputs.shape
    x = jnp.transpose(inputs, (0, 2, 3, 1))
    flat = x.reshape(-1, _D)
    xsq = jnp.sum(flat ** 2, axis=1, keepdims=True)
    esq = jnp.sum(embedding_weight ** 2, axis=1)[None, :]

    idx2, msum = _argmin_dist(flat, embedding_weight, xsq, esq)
    idx = idx2[:, 0]

    q_flat = _sc_gather(embedding_weight, idx)
    quantized_st = jnp.transpose(q_flat.reshape(B, H, W, C), (0, 3, 1, 2))
    loss = msum[0, 0] * (1.25 / (_N * _D))
    return quantized_st, loss, idx.reshape(B, H, W)
